# Initial kernel scaffold; baseline (speedup 1.0000x reference)
import jax
import jax.numpy as jnp
from jax.experimental import pallas as pl


def kernel(atomic_numbers, pos, batch, embed, P, W1, b1, W2, b2):
    raise NotImplementedError("write your pallas kernel here")


# speedup vs baseline: 10.0197x; 10.0197x over previous
import functools

import jax
import jax.numpy as jnp
from jax import lax
from jax.experimental import pallas as pl
from jax.experimental.pallas import tpu as pltpu
from jax.experimental.pallas import tpu_sc as plsc

N_AT = 32768
N_MOLS = 16
DIM = 256
TBLW = 128
TBL = N_MOLS * TBLW

_NC = 2
_NS = 16
_L = 16
_NW = _NC * _NS
_CHUNK = N_AT // _NW

@functools.cache
def _make_sc_hist():
    mesh = plsc.VectorSubcoreMesh(
        core_axis_name="c", subcore_axis_name="s", num_cores=_NC
    )
    return functools.partial(
        pl.kernel,
        mesh=mesh,
        compiler_params=pltpu.CompilerParams(needs_layout_passes=False),
        out_type=jax.ShapeDtypeStruct((_NW, TBL), jnp.float32),
        scratch_types=[
            pltpu.VMEM((_CHUNK,), jnp.int32),
            pltpu.VMEM((_CHUNK,), jnp.int32),
            pltpu.VMEM((_CHUNK,), jnp.float32),
            pltpu.VMEM((_CHUNK,), jnp.float32),
            pltpu.VMEM((_CHUNK,), jnp.float32),
            pltpu.VMEM((TBL,), jnp.float32),
        ],
    )(_sc_hist_body)


def _sc_hist_body(a_hbm, b_hbm, x_hbm, y_hbm, z_hbm, out_hbm,
                  a_v, b_v, x_v, y_v, z_v, acc):
    wid = lax.axis_index("s") * _NC + lax.axis_index("c")
    base = wid * _CHUNK
    pltpu.sync_copy(a_hbm.at[pl.ds(base, _CHUNK)], a_v)
    pltpu.sync_copy(b_hbm.at[pl.ds(base, _CHUNK)], b_v)
    pltpu.sync_copy(x_hbm.at[pl.ds(base, _CHUNK)], x_v)
    pltpu.sync_copy(y_hbm.at[pl.ds(base, _CHUNK)], y_v)
    pltpu.sync_copy(z_hbm.at[pl.ds(base, _CHUNK)], z_v)

    zeros = jnp.zeros((_L,), jnp.float32)

    def zbody(i, c):
        acc[pl.ds(i * _L, _L)] = zeros
        return c

    lax.fori_loop(0, TBL // _L, zbody, 0)

    ones = jnp.ones((_L,), jnp.float32)

    def body(i, c):
        off = i * _L
        av = a_v[pl.ds(off, _L)]
        bv = b_v[pl.ds(off, _L)]
        x = x_v[pl.ds(off, _L)]
        y = y_v[pl.ds(off, _L)]
        z = z_v[pl.ds(off, _L)]
        slot = bv * TBLW
        plsc.addupdate_scatter(acc, [slot + av], ones)
        plsc.addupdate_scatter(acc, [slot + 100], x)
        plsc.addupdate_scatter(acc, [slot + 101], y)
        plsc.addupdate_scatter(acc, [slot + 102], z)
        return c

    lax.fori_loop(0, _CHUNK // _L, body, 0)

    pltpu.sync_copy(acc, out_hbm.at[wid])


def _round_bf16(x):
    u = lax.bitcast_convert_type(x, jnp.uint32)
    u = u + jnp.uint32(0x7FFF) + ((u >> 16) & jnp.uint32(1))
    u = u & jnp.uint32(0xFFFF0000)
    return lax.bitcast_convert_type(u, jnp.float32)


def _erf(x):
    s = jnp.sign(x)
    ax = jnp.abs(x)
    t = 1.0 / (1.0 + 0.3275911 * ax)
    poly = t * (0.254829592 + t * (-0.284496736 + t * (1.421413741
           + t * (-1.453152027 + t * 1.061405429))))
    return s * (1.0 - poly * jnp.exp(-ax * ax))


def _tc_mlp_body(part_ref, wext_ref, w1_ref, b1_ref, w2_ref, b2_ref, out_ref):
    hi = lax.Precision.HIGHEST
    s = jnp.sum(part_ref[...], axis=0)
    red = jnp.dot(s, wext_ref[...], precision=hi,
                  preferred_element_type=jnp.float32)
    h = jnp.dot(red, w1_ref[...],
                preferred_element_type=jnp.float32) + b1_ref[...]
    g = h * 0.5 * (1.0 + _erf(h * 0.7071067811865476))
    out_ref[...] = (
        jnp.dot(g, w2_ref[...],
                preferred_element_type=jnp.float32) + b2_ref[...]
    )


def kernel(atomic_numbers, pos, batch, embed, P, W1, b1, W2, b2):
    a = atomic_numbers.astype(jnp.int32)
    b = batch.astype(jnp.int32)
    posf = _round_bf16(pos.astype(jnp.float32))
    px, py, pz = posf[:, 0], posf[:, 1], posf[:, 2]

    part = _make_sc_hist()(a, b, px, py, pz)
    part3 = part.reshape(_NW, N_MOLS, TBLW)

    wext = jnp.zeros((TBLW, DIM), jnp.float32)
    wext = lax.dynamic_update_slice(wext, embed, (0, 0))
    wext = lax.dynamic_update_slice(
        wext, _round_bf16(P.astype(jnp.float32)), (100, 0))
    b1r = b1.reshape(1, DIM)
    w2pad = jnp.pad(W2, ((0, 0), (0, TBLW - W2.shape[1])))
    b2pad = jnp.pad(b2, (0, TBLW - b2.shape[0])).reshape(1, TBLW)

    out = pl.pallas_call(
        _tc_mlp_body,
        out_shape=jax.ShapeDtypeStruct((N_MOLS, TBLW), jnp.float32),
    )(part3, wext, W1, b1r, w2pad, b2pad)
    return out[:, : W2.shape[1]]


# speedup vs baseline: 10.5261x; 1.0505x over previous
import functools

import jax
import jax.numpy as jnp
from jax import lax
from jax.experimental import pallas as pl
from jax.experimental.pallas import tpu as pltpu
from jax.experimental.pallas import tpu_sc as plsc

N_AT = 32768
N_MOLS = 16
DIM = 256
TBLW = 128
TBL = N_MOLS * TBLW

_NC = 2
_NS = 16
_L = 16
_NW = _NC * _NS
_CHUNK = N_AT // _NW

@functools.cache
def _make_sc_hist():
    mesh = plsc.VectorSubcoreMesh(
        core_axis_name="c", subcore_axis_name="s", num_cores=_NC
    )
    return functools.partial(
        pl.kernel,
        mesh=mesh,
        compiler_params=pltpu.CompilerParams(needs_layout_passes=False),
        out_type=jax.ShapeDtypeStruct((_NW, TBL), jnp.float32),
        scratch_types=[
            pltpu.VMEM((_CHUNK,), jnp.int32),
            pltpu.VMEM((_CHUNK,), jnp.int32),
            pltpu.VMEM((_CHUNK,), jnp.float32),
            pltpu.VMEM((_CHUNK,), jnp.float32),
            pltpu.VMEM((_CHUNK,), jnp.float32),
            pltpu.VMEM((TBL,), jnp.float32),
            pltpu.VMEM((_L * 48,), jnp.float32),
        ],
    )(_sc_hist_body)


def _sc_hist_body(a_hbm, b_hbm, x_hbm, y_hbm, z_hbm, out_hbm,
                  a_v, b_v, x_v, y_v, z_v, acc, stripe):
    wid = lax.axis_index("s") * _NC + lax.axis_index("c")
    base = wid * _CHUNK
    pltpu.sync_copy(a_hbm.at[pl.ds(base, _CHUNK)], a_v)
    pltpu.sync_copy(b_hbm.at[pl.ds(base, _CHUNK)], b_v)
    pltpu.sync_copy(x_hbm.at[pl.ds(base, _CHUNK)], x_v)
    pltpu.sync_copy(y_hbm.at[pl.ds(base, _CHUNK)], y_v)
    pltpu.sync_copy(z_hbm.at[pl.ds(base, _CHUNK)], z_v)

    zeros = jnp.zeros((_L,), jnp.float32)

    def zbody(i, c):
        acc[pl.ds(i * _L, _L)] = zeros
        return c

    lax.fori_loop(0, TBL // _L, zbody, 0)

    def zbody2(i, c):
        stripe[pl.ds(i * _L, _L)] = zeros
        return c

    lax.fori_loop(0, _L * 48 // _L, zbody2, 0)

    ones = jnp.ones((_L,), jnp.float32)
    iota = lax.iota(jnp.int32, _L)
    lane48 = iota * 48

    def body(i, c):
        off = i * _L
        av = a_v[pl.ds(off, _L)]
        bv = b_v[pl.ds(off, _L)]
        x = x_v[pl.ds(off, _L)]
        y = y_v[pl.ds(off, _L)]
        z = z_v[pl.ds(off, _L)]
        plsc.addupdate_scatter(acc, [bv * TBLW + av], ones)
        sp = lane48 + bv
        plsc.addupdate_scatter(stripe, [sp], x)
        plsc.addupdate_scatter(stripe, [sp + 16], y)
        plsc.addupdate_scatter(stripe, [sp + 32], z)
        return c

    lax.fori_loop(0, _CHUNK // _L, body, 0)

    for d in range(3):
        v = stripe[pl.ds(d * _L, _L)]
        for l in range(1, _L):
            v = v + stripe[pl.ds(l * 48 + d * _L, _L)]
        plsc.store_scatter(acc, [iota * TBLW + (100 + d)], v)

    pltpu.sync_copy(acc, out_hbm.at[wid])


def _round_bf16(x):
    u = lax.bitcast_convert_type(x, jnp.uint32)
    u = u + jnp.uint32(0x7FFF) + ((u >> 16) & jnp.uint32(1))
    u = u & jnp.uint32(0xFFFF0000)
    return lax.bitcast_convert_type(u, jnp.float32)


def _erf(x):
    s = jnp.sign(x)
    ax = jnp.abs(x)
    t = 1.0 / (1.0 + 0.3275911 * ax)
    poly = t * (0.254829592 + t * (-0.284496736 + t * (1.421413741
           + t * (-1.453152027 + t * 1.061405429))))
    return s * (1.0 - poly * jnp.exp(-ax * ax))


def _tc_mlp_body(part_ref, wext_ref, w1_ref, b1_ref, w2_ref, b2_ref, out_ref):
    hi = lax.Precision.HIGHEST
    s = jnp.sum(part_ref[...], axis=0)
    red = jnp.dot(s, wext_ref[...], precision=hi,
                  preferred_element_type=jnp.float32)
    h = jnp.dot(red, w1_ref[...],
                preferred_element_type=jnp.float32) + b1_ref[...]
    g = h * 0.5 * (1.0 + _erf(h * 0.7071067811865476))
    out_ref[...] = (
        jnp.dot(g, w2_ref[...],
                preferred_element_type=jnp.float32) + b2_ref[...]
    )


def kernel(atomic_numbers, pos, batch, embed, P, W1, b1, W2, b2):
    a = atomic_numbers.astype(jnp.int32)
    b = batch.astype(jnp.int32)
    posf = _round_bf16(pos.astype(jnp.float32))
    px, py, pz = posf[:, 0], posf[:, 1], posf[:, 2]

    part = _make_sc_hist()(a, b, px, py, pz)
    part3 = part.reshape(_NW, N_MOLS, TBLW)

    wext = jnp.zeros((TBLW, DIM), jnp.float32)
    wext = lax.dynamic_update_slice(wext, embed, (0, 0))
    wext = lax.dynamic_update_slice(
        wext, _round_bf16(P.astype(jnp.float32)), (100, 0))
    b1r = b1.reshape(1, DIM)
    w2pad = jnp.pad(W2, ((0, 0), (0, TBLW - W2.shape[1])))
    b2pad = jnp.pad(b2, (0, TBLW - b2.shape[0])).reshape(1, TBLW)

    out = pl.pallas_call(
        _tc_mlp_body,
        out_shape=jax.ShapeDtypeStruct((N_MOLS, TBLW), jnp.float32),
    )(part3, wext, W1, b1r, w2pad, b2pad)
    return out[:, : W2.shape[1]]
